# Initial kernel scaffold; baseline (speedup 1.0000x reference)
import jax
import jax.numpy as jnp
from jax.experimental import pallas as pl


def kernel(h, batch):
    raise NotImplementedError("write your pallas kernel here")


# speedup vs baseline: 4.0881x; 4.0881x over previous
import functools

import jax
import jax.numpy as jnp
from jax import lax
from jax.experimental import pallas as pl
from jax.experimental.pallas import tpu as pltpu
from jax.experimental.pallas import tpu_sc as plsc

N = 100000
H = 128
S = 512
NC = 2
NS = 16
NW = NC * NS
CHUNK = 112
BASE = 3136
LAST = N - (NW - 1) * BASE
NCH = BASE // CHUNK
NCH_LAST = LAST // CHUNK
TAIL = LAST - NCH_LAST * CHUNK
ROWS_PER_TILE = S // NS


def _pool_body(h_hbm, b_hbm, z128_hbm, z16_hbm, ones_hbm, part_out, cnt_out,
               acc_sh, cnt_sh, rows_v, idx_v, ones_v, trows_v, tidx_v,
               tones_v):
    c = lax.axis_index("c")
    s = lax.axis_index("s")
    wid = c * NS + s
    base = wid * BASE

    pltpu.sync_copy(z128_hbm.at[pl.ds(s * ROWS_PER_TILE, ROWS_PER_TILE)],
                    acc_sh.at[pl.ds(s * ROWS_PER_TILE, ROWS_PER_TILE)])
    pltpu.sync_copy(z16_hbm.at[pl.ds(s * ROWS_PER_TILE, ROWS_PER_TILE)],
                    cnt_sh.at[pl.ds(s * ROWS_PER_TILE, ROWS_PER_TILE)])

    pltpu.sync_copy(ones_hbm.at[pl.ds(0, CHUNK)], ones_v)
    pltpu.sync_copy(ones_hbm.at[pl.ds(0, TAIL)], tones_v)

    plsc.subcore_barrier()

    nch = jnp.where(wid == NW - 1, NCH_LAST, NCH)

    def _chunk(i, _):
        off = base + i * CHUNK
        pltpu.sync_copy(b_hbm.at[pl.ds(off, CHUNK)], idx_v)
        pltpu.sync_copy(h_hbm.at[pl.ds(off, CHUNK)], rows_v)
        pltpu.sync_copy(rows_v, acc_sh.at[idx_v], add=True)
        pltpu.sync_copy(ones_v, cnt_sh.at[idx_v], add=True)
        return _

    lax.fori_loop(0, nch, _chunk, 0)

    @pl.when(wid == NW - 1)
    def _tail():
        off = base + NCH_LAST * CHUNK
        pltpu.sync_copy(b_hbm.at[pl.ds(off, TAIL)], tidx_v)
        pltpu.sync_copy(h_hbm.at[pl.ds(off, TAIL)], trows_v)
        pltpu.sync_copy(trows_v, acc_sh.at[tidx_v], add=True)
        pltpu.sync_copy(tones_v, cnt_sh.at[tidx_v], add=True)

    plsc.subcore_barrier()

    r0 = s * ROWS_PER_TILE
    pltpu.sync_copy(acc_sh.at[pl.ds(r0, ROWS_PER_TILE)],
                    part_out.at[c, pl.ds(r0, ROWS_PER_TILE)])
    pltpu.sync_copy(cnt_sh.at[pl.ds(r0, ROWS_PER_TILE)],
                    cnt_out.at[c, pl.ds(r0, ROWS_PER_TILE)])


@jax.jit
def _sc_pool(h, b32, z128, z16, ones):
    mesh = plsc.VectorSubcoreMesh(core_axis_name="c", subcore_axis_name="s")
    f = pl.kernel(
        _pool_body,
        out_type=(
            jax.ShapeDtypeStruct((NC, S, H), jnp.float32),
            jax.ShapeDtypeStruct((NC, S, H), jnp.float32),
        ),
        mesh=mesh,
        scratch_types=[
            pltpu.VMEM_SHARED((S, H), jnp.float32),
            pltpu.VMEM_SHARED((S, H), jnp.float32),
            pltpu.VMEM((CHUNK, H), jnp.float32),
            pltpu.VMEM((CHUNK,), jnp.int32),
            pltpu.VMEM((CHUNK, H), jnp.float32),
            pltpu.VMEM((TAIL, H), jnp.float32),
            pltpu.VMEM((TAIL,), jnp.int32),
            pltpu.VMEM((TAIL, H), jnp.float32),
        ],
    )
    return f(h, b32, z128, z16, ones)


def _combine_body(p_ref, c_ref, o_ref):
    p = p_ref[0] + p_ref[1]
    cnt = c_ref[0] + c_ref[1]
    cnt0 = jnp.maximum(cnt[:, 0:1], 1.0)
    o_ref[...] = p / cnt0


@jax.jit
def _combine(part, cnt):
    return pl.pallas_call(
        _combine_body,
        out_shape=jax.ShapeDtypeStruct((S, H), jnp.float32),
    )(part, cnt)


def kernel(h, batch):
    b32 = batch.astype(jnp.int32)
    z128 = jnp.zeros((S, H), jnp.float32)
    z16 = jnp.zeros((S, H), jnp.float32)
    ones = jnp.ones((CHUNK, H), jnp.float32)
    part, cnt = _sc_pool(h, b32, z128, z16, ones)
    return _combine(part, cnt)


# speedup vs baseline: 5.2401x; 1.2818x over previous
import jax
import jax.numpy as jnp
from jax import lax
from jax.experimental import pallas as pl
from jax.experimental.pallas import tpu as pltpu
from jax.experimental.pallas import tpu_sc as plsc

N = 100000
H = 128
S = 512
NC = 2
NS = 16
NW = NC * NS
CHUNK = 112
BASE = 3136
LAST = N - (NW - 1) * BASE
NCH = BASE // CHUNK
NCH_LAST = LAST // CHUNK
TAIL = LAST - NCH_LAST * CHUNK
ROWS_PER_TILE = S // NS


def _pool_body(h_hbm, b_hbm, z128_hbm, z512_hbm, part_out, cnt_out,
               acc_sh, rows_v, idx_v, cnt_v, trows_v, tidx_v):
    c = lax.axis_index("c")
    s = lax.axis_index("s")
    wid = c * NS + s
    base = wid * BASE

    pltpu.sync_copy(z128_hbm.at[pl.ds(s * ROWS_PER_TILE, ROWS_PER_TILE)],
                    acc_sh.at[pl.ds(s * ROWS_PER_TILE, ROWS_PER_TILE)])
    pltpu.sync_copy(z512_hbm, cnt_v)

    plsc.subcore_barrier()

    nch = jnp.where(wid == NW - 1, NCH_LAST, NCH)
    ones16 = jnp.full((16,), 1.0, jnp.float32)

    def _chunk(i, _):
        off = base + i * CHUNK
        pltpu.sync_copy(b_hbm.at[pl.ds(off, CHUNK)], idx_v)
        pltpu.sync_copy(h_hbm.at[pl.ds(off, CHUNK)], rows_v)
        pltpu.sync_copy(rows_v, acc_sh.at[idx_v], add=True)
        for k in range(CHUNK // 16):
            plsc.addupdate_scatter(cnt_v, [idx_v[pl.ds(16 * k, 16)]], ones16)
        return _

    lax.fori_loop(0, nch, _chunk, 0)

    @pl.when(wid == NW - 1)
    def _tail():
        off = base + NCH_LAST * CHUNK
        pltpu.sync_copy(b_hbm.at[pl.ds(off, TAIL)], tidx_v)
        pltpu.sync_copy(h_hbm.at[pl.ds(off, TAIL)], trows_v)
        pltpu.sync_copy(trows_v, acc_sh.at[tidx_v], add=True)
        for k in range(TAIL // 16):
            plsc.addupdate_scatter(cnt_v, [tidx_v[pl.ds(16 * k, 16)]], ones16)

    plsc.subcore_barrier()

    r0 = s * ROWS_PER_TILE
    pltpu.sync_copy(acc_sh.at[pl.ds(r0, ROWS_PER_TILE)],
                    part_out.at[c, pl.ds(r0, ROWS_PER_TILE)])
    pltpu.sync_copy(cnt_v, cnt_out.at[c, s])


@jax.jit
def _sc_pool(h, b32, z128, z512):
    mesh = plsc.VectorSubcoreMesh(core_axis_name="c", subcore_axis_name="s")
    f = pl.kernel(
        _pool_body,
        out_type=(
            jax.ShapeDtypeStruct((NC, S, H), jnp.float32),
            jax.ShapeDtypeStruct((NC, NS, S), jnp.float32),
        ),
        mesh=mesh,
        compiler_params=pltpu.CompilerParams(needs_layout_passes=False),
        scratch_types=[
            pltpu.VMEM_SHARED((S, H), jnp.float32),
            pltpu.VMEM((CHUNK, H), jnp.float32),
            pltpu.VMEM((CHUNK,), jnp.int32),
            pltpu.VMEM((S,), jnp.float32),
            pltpu.VMEM((TAIL, H), jnp.float32),
            pltpu.VMEM((TAIL,), jnp.int32),
        ],
    )
    return f(h, b32, z128, z512)


def _combine_body(p_ref, c_ref, o_ref):
    p = p_ref[0] + p_ref[1]
    cnt = jnp.sum(c_ref[...], axis=(0, 1))
    cnt = jnp.maximum(cnt, 1.0)
    o_ref[...] = p / cnt.reshape(S, 1)


@jax.jit
def _combine(part, cnt):
    return pl.pallas_call(
        _combine_body,
        out_shape=jax.ShapeDtypeStruct((S, H), jnp.float32),
    )(part, cnt)


def kernel(h, batch):
    b32 = batch.astype(jnp.int32)
    z128 = jnp.zeros((S, H), jnp.float32)
    z512 = jnp.zeros((S,), jnp.float32)
    part, cnt = _sc_pool(h, b32, z128, z512)
    return _combine(part, cnt)


# speedup vs baseline: 7.2536x; 1.3842x over previous
import jax
import jax.numpy as jnp
from jax import lax
from jax.experimental import pallas as pl
from jax.experimental.pallas import tpu as pltpu
from jax.experimental.pallas import tpu_sc as plsc

N = 100000
H = 128
S = 512
NC = 2
NS = 16
NW = NC * NS
CHUNK = 112
BASE = 3136
LAST = N - (NW - 1) * BASE
NCH = BASE // CHUNK
NCH_LAST = LAST // CHUNK
TAIL = LAST - NCH_LAST * CHUNK
ROWS_PER_TILE = S // NS
GCH = NW * NCH


def _pool_body(h_hbm, b_hbm, b2_hbm, z128_hbm, z512_hbm, part_out, cnt_out,
               acc_sh, rows0_v, rows1_v, idx2_v, cnt_v, trows_v, tidx_v,
               sem0, sem1):
    c = lax.axis_index("c")
    s = lax.axis_index("s")
    wid = c * NS + s
    base = wid * BASE

    pltpu.sync_copy(z128_hbm.at[pl.ds(s * ROWS_PER_TILE, ROWS_PER_TILE)],
                    acc_sh.at[pl.ds(s * ROWS_PER_TILE, ROWS_PER_TILE)])
    pltpu.sync_copy(z512_hbm, cnt_v)
    pltpu.sync_copy(b2_hbm.at[wid], idx2_v)

    plsc.subcore_barrier()

    nch2 = jnp.where(wid == NW - 1, NCH_LAST // 2, NCH // 2)
    ones16 = jnp.full((16,), 1.0, jnp.float32)

    def _off(i):
        return base + i * CHUNK

    def _counts(i):
        for k in range(CHUNK // 16):
            plsc.addupdate_scatter(cnt_v, [idx2_v[i, pl.ds(16 * k, 16)]],
                                   ones16)

    pltpu.async_copy(h_hbm.at[pl.ds(base, CHUNK)], rows0_v, sem0)

    def _pair(j, _):
        i0 = 2 * j
        i1 = 2 * j + 1
        pltpu.async_copy(h_hbm.at[pl.ds(_off(i1), CHUNK)], rows1_v, sem1)
        pltpu.make_async_copy(h_hbm.at[pl.ds(_off(i0), CHUNK)], rows0_v,
                              sem0).wait()
        pltpu.sync_copy(rows0_v, acc_sh.at[idx2_v.at[i0]], add=True)
        _counts(i0)
        off2 = jnp.minimum(_off(i0 + 2), N - CHUNK)
        pltpu.async_copy(h_hbm.at[pl.ds(off2, CHUNK)], rows0_v, sem0)
        pltpu.make_async_copy(h_hbm.at[pl.ds(_off(i1), CHUNK)], rows1_v,
                              sem1).wait()
        pltpu.sync_copy(rows1_v, acc_sh.at[idx2_v.at[i1]], add=True)
        _counts(i1)
        return _

    lax.fori_loop(0, nch2, _pair, 0)

    pltpu.make_async_copy(h_hbm.at[pl.ds(0, CHUNK)], rows0_v, sem0).wait()

    @pl.when(wid == NW - 1)
    def _tail():
        off = base + NCH_LAST * CHUNK
        pltpu.sync_copy(b_hbm.at[pl.ds(off, TAIL)], tidx_v)
        pltpu.sync_copy(h_hbm.at[pl.ds(off, TAIL)], trows_v)
        pltpu.sync_copy(trows_v, acc_sh.at[tidx_v], add=True)
        for k in range(TAIL // 16):
            plsc.addupdate_scatter(cnt_v, [tidx_v[pl.ds(16 * k, 16)]], ones16)

    plsc.subcore_barrier()

    r0 = s * ROWS_PER_TILE
    pltpu.sync_copy(acc_sh.at[pl.ds(r0, ROWS_PER_TILE)],
                    part_out.at[c, pl.ds(r0, ROWS_PER_TILE)])
    pltpu.sync_copy(cnt_v, cnt_out.at[c, s])


@jax.jit
def _sc_pool(h, b32, b2d, z128, z512):
    mesh = plsc.VectorSubcoreMesh(core_axis_name="c", subcore_axis_name="s")
    f = pl.kernel(
        _pool_body,
        out_type=(
            jax.ShapeDtypeStruct((NC, S, H), jnp.float32),
            jax.ShapeDtypeStruct((NC, NS, S), jnp.float32),
        ),
        mesh=mesh,
        compiler_params=pltpu.CompilerParams(needs_layout_passes=False),
        scratch_types=[
            pltpu.VMEM_SHARED((S, H), jnp.float32),
            pltpu.VMEM((CHUNK, H), jnp.float32),
            pltpu.VMEM((CHUNK, H), jnp.float32),
            pltpu.VMEM((NCH, CHUNK), jnp.int32),
            pltpu.VMEM((S,), jnp.float32),
            pltpu.VMEM((TAIL, H), jnp.float32),
            pltpu.VMEM((TAIL,), jnp.int32),
            pltpu.SemaphoreType.DMA,
            pltpu.SemaphoreType.DMA,
        ],
    )
    return f(h, b32, b2d, z128, z512)


def _combine_body(p_ref, c_ref, o_ref):
    p = p_ref[0] + p_ref[1]
    cnt = jnp.sum(c_ref[...], axis=(0, 1))
    cnt = jnp.maximum(cnt, 1.0)
    o_ref[...] = p / cnt.reshape(S, 1)


@jax.jit
def _combine(part, cnt):
    return pl.pallas_call(
        _combine_body,
        out_shape=jax.ShapeDtypeStruct((S, H), jnp.float32),
    )(part, cnt)


def kernel(h, batch):
    b32 = batch.astype(jnp.int32)
    b2d = jnp.concatenate(
        [b32, jnp.zeros((GCH * CHUNK - N,), jnp.int32)]).reshape(
            NW, NCH, CHUNK)
    z128 = jnp.zeros((S, H), jnp.float32)
    z512 = jnp.zeros((S,), jnp.float32)
    part, cnt = _sc_pool(h, b32, b2d, z128, z512)
    return _combine(part, cnt)
